# baseline (speedup 1.0000x reference)
import jax
import jax.numpy as jnp
from jax import lax
from jax.experimental import pallas as pl
from jax.experimental.pallas import tpu as pltpu
from jax.experimental.pallas import tpu_sc as plsc

_B = 1024
_L = 200
_V = 100000
_LN = 16
_NQ = 4
_BG = 128
_QV = _V // _NQ
_CV = 200
_NC = _QV // _CV
_CAP = 16
_CAPL = 800
_IDSP = 16 * _L
_M200 = 5243


def _idf_body(ids_hbm, w_hbm, out_hbm, ids_v, w_v, seg_v, bkt_v, cnt_v,
              buf_a, buf_b, sem_a, sem_b, sem_i):
    wid = lax.axis_index("s") * 2 + lax.axis_index("c")
    g = wid // _NQ
    q = wid % _NQ
    q0 = q * _QV
    gofs = g * _BG
    ids_base = g * (_BG * _L)

    z16f = jnp.zeros((_LN,), jnp.float32)
    z16i = jnp.zeros((_LN,), jnp.int32)
    lanes = lax.iota(jnp.int32, _LN)

    def zero_buf(buf):
        def zb(i, _):
            buf[i // 8, pl.ds((i % 8) * _LN, _LN)] = z16f
            return 0
        lax.fori_loop(0, (_CV * _BG) // _LN, zb, 0)

    zero_buf(buf_a)
    zero_buf(buf_b)

    def zc(i, _):
        cnt_v[pl.ds(i * _LN, _LN)] = z16i
        return 0

    lax.fori_loop(0, _NC, zc, 0)

    pltpu.sync_copy(w_hbm.at[pl.ds(q0, _QV)], w_v)

    def ids_half(p):
        return ids_v.at[pl.ds((p % 2) * _IDSP, _IDSP)]

    pltpu.async_copy(ids_hbm.at[pl.ds(ids_base, _IDSP)], ids_half(0), sem_i)
    cnt1 = z16i
    for piece in range(8):
        pltpu.make_async_copy(ids_hbm.at[pl.ds(0, _IDSP)], ids_half(piece),
                              sem_i).wait()
        if piece < 7:
            pltpu.async_copy(
                ids_hbm.at[pl.ds(ids_base + (piece + 1) * _IDSP, _IDSP)],
                ids_half(piece + 1), sem_i)
        hofs = (piece % 2) * _IDSP

        def p1(i, cnt):
            for u in range(4):
                vec = i * 4 + u
                v = ids_v[pl.ds(hofs + vec * _LN, _LN)]
                t = (piece * _IDSP + vec * _LN) + lanes
                bloc = lax.shift_right_logical(t * _M200, 20)
                m = (v >= q0) & (v < q0 + _QV)
                vq = v - q0
                rec = vq * _BG + bloc
                ok = m & (cnt < _CAPL)
                plsc.store_scatter(seg_v, [cnt * _LN + lanes], rec, mask=ok)
                cnt = cnt + m.astype(jnp.int32)
            return cnt

        cnt1 = lax.fori_loop(0, _IDSP // _LN // 4, p1, cnt1)
    n1 = jnp.max(cnt1)

    def p2(j, _):
        rec = seg_v[pl.ds(j * _LN, _LN)]
        mj = cnt1 > j
        vq = lax.shift_right_logical(rec, 7)
        bloc = rec & (_BG - 1)
        c = lax.shift_right_logical(vq * _M200, 20)
        vloc = vq - c * _CV
        slot = c * _LN + lanes
        cnt = plsc.load_gather(cnt_v, [slot], mask=mj)
        ok = mj & (cnt < _CAP)
        off = c * (_CAP * _LN) + cnt * _LN + lanes
        plsc.store_scatter(bkt_v, [off], vloc * _BG + bloc, mask=ok)
        plsc.store_scatter(cnt_v, [slot], cnt + 1, mask=mj)
        return 0

    lax.fori_loop(0, jnp.minimum(n1, _CAPL), p2, 0)

    def poison():
        big = jnp.full((_LN,), _CAP + 1, jnp.int32)

        def pz(i, _):
            cnt_v[pl.ds(i * _LN, _LN)] = big
            return 0

        lax.fori_loop(0, _NC, pz, 0)

    pl.when(n1 > _CAPL)(poison)

    def chunk_counts(c):
        cnts = cnt_v[pl.ds(c * _LN, _LN)]
        mx = jnp.max(cnts)
        return cnts, mx

    def bucket_pass(c, buf, cnts, mx, write_values):
        def jb(j, _):
            rec = bkt_v[pl.ds(c * (_CAP * _LN) + j * _LN, _LN)]
            m = cnts > j
            vloc = lax.shift_right_logical(rec, 7)
            bloc = rec & (_BG - 1)
            if write_values:
                val = plsc.load_gather(w_v, [c * _CV + vloc], mask=m)
            else:
                val = z16f
            plsc.store_scatter(buf, [vloc, bloc], val, mask=m)
            return 0

        lax.fori_loop(0, mx, jb, 0)

    def rescan_scatter(c, buf):
        for piece in range(8):
            pltpu.sync_copy(
                ids_hbm.at[pl.ds(ids_base + piece * _IDSP, _IDSP)],
                ids_v.at[pl.ds(0, _IDSP)])

            def rs(i, _):
                v = ids_v[pl.ds(i * _LN, _LN)]
                t = (piece * _IDSP + i * _LN) + lanes
                bloc = lax.shift_right_logical(t * _M200, 20)
                vs = q0 + c * _CV
                m = (v >= vs) & (v < vs + _CV)
                vloc = v - vs
                val = plsc.load_gather(w_v, [v - q0], mask=m)
                plsc.store_scatter(buf, [vloc, bloc], val, mask=m)
                return 0

            lax.fori_loop(0, _IDSP // _LN, rs, 0)

    def scatter_chunk(c, buf):
        cnts, mx = chunk_counts(c)
        pl.when(mx <= _CAP)(lambda: bucket_pass(c, buf, cnts, mx, True))
        pl.when(mx > _CAP)(lambda: rescan_scatter(c, buf))

    def restore_chunk(c, buf):
        cnts, mx = chunk_counts(c)
        pl.when(mx <= _CAP)(lambda: bucket_pass(c, buf, cnts, mx, False))
        pl.when(mx > _CAP)(lambda: zero_buf(buf))

    def out_block(c):
        return out_hbm.at[pl.ds(q0 + c * _CV, _CV), pl.ds(gofs, _BG)]

    def start_stream(c, buf, sem):
        pltpu.async_copy(buf, out_block(c), sem)

    def wait_stream(buf, sem):
        pltpu.make_async_copy(out_hbm.at[pl.ds(0, _CV), pl.ds(0, _BG)],
                              buf, sem).wait()

    scatter_chunk(0, buf_a)
    start_stream(0, buf_a, sem_a)
    scatter_chunk(1, buf_b)
    start_stream(1, buf_b, sem_b)

    def pair(k, _):
        c = 2 * k
        wait_stream(buf_a, sem_a)
        restore_chunk(c - 2, buf_a)
        scatter_chunk(c, buf_a)
        start_stream(c, buf_a, sem_a)
        wait_stream(buf_b, sem_b)
        restore_chunk(c - 1, buf_b)
        scatter_chunk(c + 1, buf_b)
        start_stream(c + 1, buf_b, sem_b)
        return 0

    lax.fori_loop(1, (_NC - 1) // 2, pair, 0)

    wait_stream(buf_a, sem_a)
    restore_chunk(_NC - 3, buf_a)
    scatter_chunk(_NC - 1, buf_a)
    start_stream(_NC - 1, buf_a, sem_a)
    wait_stream(buf_b, sem_b)
    wait_stream(buf_a, sem_a)


@jax.jit
def _idf(input_ids, weight):
    mesh = plsc.VectorSubcoreMesh(core_axis_name="c", subcore_axis_name="s")
    out_t = pl.kernel(
        _idf_body,
        out_type=jax.ShapeDtypeStruct((_V, _B), jnp.float32),
        mesh=mesh,
        compiler_params=pltpu.CompilerParams(needs_layout_passes=False),
        scratch_types=[
            pltpu.VMEM((2 * _IDSP,), jnp.int32),
            pltpu.VMEM((_QV,), jnp.float32),
            pltpu.VMEM((_CAPL * _LN,), jnp.int32),
            pltpu.VMEM((_NC * _CAP * _LN,), jnp.int32),
            pltpu.VMEM((_NC * _LN,), jnp.int32),
            pltpu.VMEM((_CV, _BG), jnp.float32),
            pltpu.VMEM((_CV, _BG), jnp.float32),
            pltpu.SemaphoreType.DMA,
            pltpu.SemaphoreType.DMA,
            pltpu.SemaphoreType.DMA,
        ],
    )(input_ids.reshape(-1), weight)
    return out_t.T


def kernel(input_ids, weight):
    return _idf(input_ids, weight)


# speedup vs baseline: 1.0202x; 1.0202x over previous
import jax
import jax.numpy as jnp
from jax import lax
from jax.experimental import pallas as pl
from jax.experimental.pallas import tpu as pltpu
from jax.experimental.pallas import tpu_sc as plsc

_B = 1024
_L = 200
_V = 100000
_LN = 16
_NQ = 4
_BG = 128
_QV = _V // _NQ
_CV = 200
_NC = _QV // _CV
_CAP = 16
_CAPL = 800
_IDSP = 16 * _L
_M200 = 5243


def _idf_body(ids_hbm, w_hbm, out_hbm, ids_v, w_v, seg_v, bkt_v, cnt_v,
              buf_a, buf_b, sem_a, sem_b, sem_i, sem_w):
    wid = lax.axis_index("s") * 2 + lax.axis_index("c")
    g = wid // _NQ
    q = wid % _NQ
    q0 = q * _QV
    gofs = g * _BG
    ids_base = g * (_BG * _L)

    z16f = jnp.zeros((_LN,), jnp.float32)
    z16i = jnp.zeros((_LN,), jnp.int32)
    lanes = lax.iota(jnp.int32, _LN)

    def ids_half(p):
        return ids_v.at[pl.ds((p % 2) * _IDSP, _IDSP)]

    pltpu.async_copy(ids_hbm.at[pl.ds(ids_base, _IDSP)], ids_half(0), sem_i)
    w_copy = pltpu.async_copy(w_hbm.at[pl.ds(q0, _QV)], w_v, sem_w)

    def zero_buf(buf):
        def zb(i, _):
            buf[i // 8, pl.ds((i % 8) * _LN, _LN)] = z16f
            return 0
        lax.fori_loop(0, (_CV * _BG) // _LN, zb, 0)

    zero_buf(buf_a)
    zero_buf(buf_b)

    def zc(i, _):
        cnt_v[pl.ds(i * _LN, _LN)] = z16i
        return 0

    lax.fori_loop(0, _NC, zc, 0)

    cnt1 = z16i
    for piece in range(8):
        pltpu.make_async_copy(ids_hbm.at[pl.ds(0, _IDSP)], ids_half(piece),
                              sem_i).wait()
        if piece < 7:
            pltpu.async_copy(
                ids_hbm.at[pl.ds(ids_base + (piece + 1) * _IDSP, _IDSP)],
                ids_half(piece + 1), sem_i)
        hofs = (piece % 2) * _IDSP

        def p1(i, cnt):
            for u in range(4):
                vec = i * 4 + u
                v = ids_v[pl.ds(hofs + vec * _LN, _LN)]
                t = (piece * _IDSP + vec * _LN) + lanes
                bloc = lax.shift_right_logical(t * _M200, 20)
                m = (v >= q0) & (v < q0 + _QV)
                vq = v - q0
                rec = vq * _BG + bloc
                ok = m & (cnt < _CAPL)
                plsc.store_scatter(seg_v, [cnt * _LN + lanes], rec, mask=ok)
                cnt = cnt + m.astype(jnp.int32)
            return cnt

        cnt1 = lax.fori_loop(0, _IDSP // _LN // 4, p1, cnt1)
    n1 = jnp.max(cnt1)

    def p2(j, _):
        rec = seg_v[pl.ds(j * _LN, _LN)]
        mj = cnt1 > j
        vq = lax.shift_right_logical(rec, 7)
        bloc = rec & (_BG - 1)
        c = lax.shift_right_logical(vq * _M200, 20)
        vloc = vq - c * _CV
        slot = c * _LN + lanes
        cnt = plsc.load_gather(cnt_v, [slot], mask=mj)
        ok = mj & (cnt < _CAP)
        off = c * (_CAP * _LN) + cnt * _LN + lanes
        plsc.store_scatter(bkt_v, [off], vloc * _BG + bloc, mask=ok)
        plsc.store_scatter(cnt_v, [slot], cnt + 1, mask=mj)
        return 0

    lax.fori_loop(0, jnp.minimum(n1, _CAPL), p2, 0)

    def poison():
        big = jnp.full((_LN,), _CAP + 1, jnp.int32)

        def pz(i, _):
            cnt_v[pl.ds(i * _LN, _LN)] = big
            return 0

        lax.fori_loop(0, _NC, pz, 0)

    pl.when(n1 > _CAPL)(poison)

    w_copy.wait()

    def chunk_counts(c):
        cnts = cnt_v[pl.ds(c * _LN, _LN)]
        mx = jnp.max(cnts)
        return cnts, mx

    def bucket_pass(c, buf, cnts, mx, write_values):
        def jb(j, _):
            rec = bkt_v[pl.ds(c * (_CAP * _LN) + j * _LN, _LN)]
            m = cnts > j
            vloc = lax.shift_right_logical(rec, 7)
            bloc = rec & (_BG - 1)
            if write_values:
                val = plsc.load_gather(w_v, [c * _CV + vloc], mask=m)
            else:
                val = z16f
            plsc.store_scatter(buf, [vloc, bloc], val, mask=m)
            return 0

        lax.fori_loop(0, mx, jb, 0)

    def rescan_scatter(c, buf):
        for piece in range(8):
            pltpu.sync_copy(
                ids_hbm.at[pl.ds(ids_base + piece * _IDSP, _IDSP)],
                ids_v.at[pl.ds(0, _IDSP)])

            def rs(i, _):
                v = ids_v[pl.ds(i * _LN, _LN)]
                t = (piece * _IDSP + i * _LN) + lanes
                bloc = lax.shift_right_logical(t * _M200, 20)
                vs = q0 + c * _CV
                m = (v >= vs) & (v < vs + _CV)
                vloc = v - vs
                val = plsc.load_gather(w_v, [v - q0], mask=m)
                plsc.store_scatter(buf, [vloc, bloc], val, mask=m)
                return 0

            lax.fori_loop(0, _IDSP // _LN, rs, 0)

    def scatter_chunk(c, buf):
        cnts, mx = chunk_counts(c)
        pl.when(mx <= _CAP)(lambda: bucket_pass(c, buf, cnts, mx, True))
        pl.when(mx > _CAP)(lambda: rescan_scatter(c, buf))

    def restore_chunk(c, buf):
        cnts, mx = chunk_counts(c)
        pl.when(mx <= _CAP)(lambda: bucket_pass(c, buf, cnts, mx, False))
        pl.when(mx > _CAP)(lambda: zero_buf(buf))

    def out_block(c):
        return out_hbm.at[pl.ds(q0 + c * _CV, _CV), pl.ds(gofs, _BG)]

    def start_stream(c, buf, sem):
        pltpu.async_copy(buf, out_block(c), sem)

    def wait_stream(buf, sem):
        pltpu.make_async_copy(out_hbm.at[pl.ds(0, _CV), pl.ds(0, _BG)],
                              buf, sem).wait()

    scatter_chunk(0, buf_a)
    start_stream(0, buf_a, sem_a)
    scatter_chunk(1, buf_b)
    start_stream(1, buf_b, sem_b)

    def pair(k, _):
        c = 2 * k
        wait_stream(buf_a, sem_a)
        restore_chunk(c - 2, buf_a)
        scatter_chunk(c, buf_a)
        start_stream(c, buf_a, sem_a)
        wait_stream(buf_b, sem_b)
        restore_chunk(c - 1, buf_b)
        scatter_chunk(c + 1, buf_b)
        start_stream(c + 1, buf_b, sem_b)
        return 0

    lax.fori_loop(1, (_NC - 1) // 2, pair, 0)

    wait_stream(buf_a, sem_a)
    restore_chunk(_NC - 3, buf_a)
    scatter_chunk(_NC - 1, buf_a)
    start_stream(_NC - 1, buf_a, sem_a)
    wait_stream(buf_b, sem_b)
    wait_stream(buf_a, sem_a)


@jax.jit
def _idf(input_ids, weight):
    mesh = plsc.VectorSubcoreMesh(core_axis_name="c", subcore_axis_name="s")
    out_t = pl.kernel(
        _idf_body,
        out_type=jax.ShapeDtypeStruct((_V, _B), jnp.float32),
        mesh=mesh,
        compiler_params=pltpu.CompilerParams(needs_layout_passes=False),
        scratch_types=[
            pltpu.VMEM((2 * _IDSP,), jnp.int32),
            pltpu.VMEM((_QV,), jnp.float32),
            pltpu.VMEM((_CAPL * _LN,), jnp.int32),
            pltpu.VMEM((_NC * _CAP * _LN,), jnp.int32),
            pltpu.VMEM((_NC * _LN,), jnp.int32),
            pltpu.VMEM((_CV, _BG), jnp.float32),
            pltpu.VMEM((_CV, _BG), jnp.float32),
            pltpu.SemaphoreType.DMA,
            pltpu.SemaphoreType.DMA,
            pltpu.SemaphoreType.DMA,
            pltpu.SemaphoreType.DMA,
        ],
    )(input_ids.reshape(-1), weight)
    return out_t.T


def kernel(input_ids, weight):
    return _idf(input_ids, weight)
